# baseline (speedup 1.0000x reference)
import functools

import jax
import jax.numpy as jnp
from jax.experimental import pallas as pl
from jax.experimental.pallas import tpu as pltpu
from jax.experimental.pallas import tpu_sc as plsc

_VOCAB = 1000000
_DIM = 128
_BATCH = 4096
_HIST = 200
_IDS = _BATCH * _HIST

_WINDOW = 128
_BB = 64
_NCHUNK = 2
_CB = _BATCH // _NCHUNK
_CIDS = _CB * _HIST


def _gather(table, flat_ids):
    mesh = plsc.VectorSubcoreMesh(core_axis_name="core",
                                  subcore_axis_name="subcore")

    @functools.partial(
        pl.kernel,
        out_type=jax.ShapeDtypeStruct((_CIDS, _DIM), jnp.float32),
        mesh=mesh,
    )
    def sc_kernel(table_hbm, ids_hbm, out_hbm):
        def body(i_vmem, o_vmem):
            pltpu.sync_copy(table_hbm.at[i_vmem.at[0]], o_vmem)

        pltpu.emit_pipeline(
            body,
            grid=(_CIDS // _WINDOW,),
            in_specs=[pl.BlockSpec((1, _WINDOW), index_map=lambda i: (0, i))],
            out_specs=[pl.BlockSpec((_WINDOW, _DIM),
                                    index_map=lambda i: (i, 0))],
            core_axis_name=("core", "subcore"),
            dimension_semantics=(pltpu.PARALLEL,),
        )(ids_hbm, out_hbm)

    return sc_kernel(table, flat_ids)


def _mlp_body(e_ref, w1t_ref, b1_ref, w2t_ref, b2_ref, s_ref, o_ref):
    h = jnp.dot(e_ref[...], w1t_ref[...],
                preferred_element_type=jnp.float32) + b1_ref[...]
    h = jnp.maximum(h, 0.0)
    hm = jnp.dot(s_ref[...], h, preferred_element_type=jnp.float32)
    o_ref[...] = jnp.dot(hm, w2t_ref[...],
                         preferred_element_type=jnp.float32) + b2_ref[...]


def _mlp(embeds, w1t, b1, w2t, b2, seg):
    grid = _CB // _BB
    return pl.pallas_call(
        _mlp_body,
        grid=(grid,),
        in_specs=[
            pl.BlockSpec((_BB * _HIST, _DIM), lambda i: (i, 0)),
            pl.BlockSpec((_DIM, _DIM), lambda i: (0, 0)),
            pl.BlockSpec((1, _DIM), lambda i: (0, 0)),
            pl.BlockSpec((_DIM, _DIM), lambda i: (0, 0)),
            pl.BlockSpec((1, _DIM), lambda i: (0, 0)),
            pl.BlockSpec((_BB, _BB * _HIST), lambda i: (0, 0)),
        ],
        out_specs=pl.BlockSpec((_BB, _DIM), lambda i: (i, 0)),
        out_shape=jax.ShapeDtypeStruct((_CB, _DIM), jnp.float32),
    )(embeds, w1t, b1, w2t, b2, seg)


def kernel(input_ids, table, W1, b1, W2, b2):
    flat_ids = input_ids.reshape(_NCHUNK, 1, _CIDS).astype(jnp.int32)
    col = jax.lax.broadcasted_iota(jnp.int32, (_BB, _BB * _HIST), 1)
    row = jax.lax.broadcasted_iota(jnp.int32, (_BB, _BB * _HIST), 0)
    seg = jnp.where(col // _HIST == row, jnp.float32(1.0 / _HIST),
                    jnp.float32(0.0))
    w1t, w2t = W1.T, W2.T
    b1r, b2r = b1.reshape(1, _DIM), b2.reshape(1, _DIM)
    outs = []
    for c in range(_NCHUNK):
        emb_c = _gather(table, flat_ids[c])
        outs.append(_mlp(emb_c, w1t, b1r, w2t, b2r, seg))
    return jnp.concatenate(outs, axis=0)


# speedup vs baseline: 1.1158x; 1.1158x over previous
import functools

import jax
import jax.numpy as jnp
from jax.experimental import pallas as pl
from jax.experimental.pallas import tpu as pltpu
from jax.experimental.pallas import tpu_sc as plsc

_VOCAB = 1000000
_DIM = 128
_BATCH = 4096
_HIST = 200
_IDS = _BATCH * _HIST

_WINDOW = 256
_BB = 64
_NCHUNK = 4
_CB = _BATCH // _NCHUNK
_CIDS = _CB * _HIST


def _gather(table, flat_ids):
    mesh = plsc.VectorSubcoreMesh(core_axis_name="core",
                                  subcore_axis_name="subcore")

    @functools.partial(
        pl.kernel,
        out_type=jax.ShapeDtypeStruct((_CIDS, _DIM), jnp.float32),
        mesh=mesh,
    )
    def sc_kernel(table_hbm, ids_hbm, out_hbm):
        def body(i_vmem, o_vmem):
            def inner(sem):
                cps = [
                    pltpu.async_copy(
                        table_hbm.at[i_vmem.at[0, pl.ds(j * 128, 128)]],
                        o_vmem.at[pl.ds(j * 128, 128)],
                        sem,
                    )
                    for j in range(_WINDOW // 128)
                ]
                for cp in cps:
                    cp.wait()

            pl.run_scoped(inner, pltpu.SemaphoreType.DMA)

        pltpu.emit_pipeline(
            body,
            grid=(_CIDS // _WINDOW,),
            in_specs=[pl.BlockSpec((1, _WINDOW), index_map=lambda i: (0, i))],
            out_specs=[pl.BlockSpec((_WINDOW, _DIM),
                                    index_map=lambda i: (i, 0))],
            core_axis_name=("core", "subcore"),
            dimension_semantics=(pltpu.PARALLEL,),
        )(ids_hbm, out_hbm)

    return sc_kernel(table, flat_ids)


def _mlp_body(e_ref, w1t_ref, b1_ref, w2t_ref, b2_ref, s_ref, o_ref):
    h = jnp.dot(e_ref[...], w1t_ref[...],
                preferred_element_type=jnp.float32) + b1_ref[...]
    h = jnp.maximum(h, 0.0)
    hm = jnp.dot(s_ref[...], h, preferred_element_type=jnp.float32)
    o_ref[...] = jnp.dot(hm, w2t_ref[...],
                         preferred_element_type=jnp.float32) + b2_ref[...]


def _mlp(embeds, w1t, b1, w2t, b2, seg):
    grid = _CB // _BB
    return pl.pallas_call(
        _mlp_body,
        grid=(grid,),
        in_specs=[
            pl.BlockSpec((_BB * _HIST, _DIM), lambda i: (i, 0)),
            pl.BlockSpec((_DIM, _DIM), lambda i: (0, 0)),
            pl.BlockSpec((1, _DIM), lambda i: (0, 0)),
            pl.BlockSpec((_DIM, _DIM), lambda i: (0, 0)),
            pl.BlockSpec((1, _DIM), lambda i: (0, 0)),
            pl.BlockSpec((_BB, _BB * _HIST), lambda i: (0, 0)),
        ],
        out_specs=pl.BlockSpec((_BB, _DIM), lambda i: (i, 0)),
        out_shape=jax.ShapeDtypeStruct((_CB, _DIM), jnp.float32),
    )(embeds, w1t, b1, w2t, b2, seg)


def kernel(input_ids, table, W1, b1, W2, b2):
    flat_ids = input_ids.reshape(_NCHUNK, 1, _CIDS).astype(jnp.int32)
    col = jax.lax.broadcasted_iota(jnp.int32, (_BB, _BB * _HIST), 1)
    row = jax.lax.broadcasted_iota(jnp.int32, (_BB, _BB * _HIST), 0)
    seg = jnp.where(col // _HIST == row, jnp.float32(1.0 / _HIST),
                    jnp.float32(0.0))
    w1t, w2t = W1.T, W2.T
    b1r, b2r = b1.reshape(1, _DIM), b2.reshape(1, _DIM)
    outs = []
    for c in range(_NCHUNK):
        emb_c = _gather(table, flat_ids[c])
        outs.append(_mlp(emb_c, w1t, b1r, w2t, b2r, seg))
    return jnp.concatenate(outs, axis=0)


# speedup vs baseline: 1.1265x; 1.0096x over previous
import functools

import jax
import jax.numpy as jnp
from jax.experimental import pallas as pl
from jax.experimental.pallas import tpu as pltpu
from jax.experimental.pallas import tpu_sc as plsc

_VOCAB = 1000000
_DIM = 128
_BATCH = 4096
_HIST = 200
_IDS = _BATCH * _HIST

_BB = 64
_NCHUNK = 4
_CB = _BATCH // _NCHUNK
_CIDS = _CB * _HIST
_TILES = 32
_GW = 128
_GPT = _CIDS // (_TILES * _GW)


def _gather(table, ids3):
    mesh = plsc.VectorSubcoreMesh(core_axis_name="core",
                                  subcore_axis_name="subcore")

    _NBUF = 5
    _LAG = 2

    @functools.partial(
        pl.kernel,
        out_type=jax.ShapeDtypeStruct((_CIDS, _DIM), jnp.float32),
        mesh=mesh,
        scratch_types=[
            pltpu.VMEM((_GPT, _GW), jnp.int32),
            pltpu.VMEM((_NBUF, _GW, _DIM), jnp.float32),
            pltpu.SemaphoreType.DMA((_NBUF,)),
            pltpu.SemaphoreType.DMA((_NBUF,)),
        ],
    )
    def sc_kernel(table_hbm, ids_hbm, out_hbm, idx_v, buf, gsem, osem):
        wid = (jax.lax.axis_index("subcore") * 2
               + jax.lax.axis_index("core"))
        pltpu.sync_copy(ids_hbm.at[wid], idx_v)
        base = wid * (_GPT * _GW)

        for j in range(_GPT + _LAG):
            b = j % _NBUF
            if j < _GPT:
                if j >= _NBUF:
                    pltpu.make_async_copy(
                        buf.at[b], out_hbm.at[pl.ds(base, _GW)],
                        osem.at[b],
                    ).wait()
                pltpu.async_copy(
                    table_hbm.at[idx_v.at[j]], buf.at[b], gsem.at[b])
            if j >= _LAG:
                jj = j - _LAG
                bb = jj % _NBUF
                pltpu.make_async_copy(
                    table_hbm.at[idx_v.at[0]], buf.at[bb], gsem.at[bb],
                ).wait()
                pltpu.async_copy(
                    buf.at[bb],
                    out_hbm.at[pl.ds(base + jj * _GW, _GW)],
                    osem.at[bb],
                )
        for j in range(_GPT - _NBUF, _GPT):
            b = j % _NBUF
            pltpu.make_async_copy(
                buf.at[b], out_hbm.at[pl.ds(base, _GW)], osem.at[b],
            ).wait()

    return sc_kernel(table, ids3)


def _mlp_body(e_ref, w1t_ref, b1_ref, w2t_ref, b2_ref, s_ref, o_ref):
    h = jnp.dot(e_ref[...], w1t_ref[...],
                preferred_element_type=jnp.float32) + b1_ref[...]
    h = jnp.maximum(h, 0.0)
    hm = jnp.dot(s_ref[...], h, preferred_element_type=jnp.float32)
    o_ref[...] = jnp.dot(hm, w2t_ref[...],
                         preferred_element_type=jnp.float32) + b2_ref[...]


def _mlp(embeds, w1t, b1, w2t, b2, seg):
    grid = _CB // _BB
    return pl.pallas_call(
        _mlp_body,
        grid=(grid,),
        in_specs=[
            pl.BlockSpec((_BB * _HIST, _DIM), lambda i: (i, 0)),
            pl.BlockSpec((_DIM, _DIM), lambda i: (0, 0)),
            pl.BlockSpec((1, _DIM), lambda i: (0, 0)),
            pl.BlockSpec((_DIM, _DIM), lambda i: (0, 0)),
            pl.BlockSpec((1, _DIM), lambda i: (0, 0)),
            pl.BlockSpec((_BB, _BB * _HIST), lambda i: (0, 0)),
        ],
        out_specs=pl.BlockSpec((_BB, _DIM), lambda i: (i, 0)),
        out_shape=jax.ShapeDtypeStruct((_CB, _DIM), jnp.float32),
    )(embeds, w1t, b1, w2t, b2, seg)


def kernel(input_ids, table, W1, b1, W2, b2):
    flat_ids = input_ids.reshape(_NCHUNK, _TILES, _GPT, _GW).astype(jnp.int32)
    col = jax.lax.broadcasted_iota(jnp.int32, (_BB, _BB * _HIST), 1)
    row = jax.lax.broadcasted_iota(jnp.int32, (_BB, _BB * _HIST), 0)
    seg = jnp.where(col // _HIST == row, jnp.float32(1.0 / _HIST),
                    jnp.float32(0.0))
    w1t, w2t = W1.T, W2.T
    b1r, b2r = b1.reshape(1, _DIM), b2.reshape(1, _DIM)
    outs = []
    for c in range(_NCHUNK):
        emb_c = _gather(table, flat_ids[c])
        outs.append(_mlp(emb_c, w1t, b1r, w2t, b2r, seg))
    return jnp.concatenate(outs, axis=0)


# speedup vs baseline: 1.1335x; 1.0062x over previous
import functools

import jax
import jax.numpy as jnp
from jax.experimental import pallas as pl
from jax.experimental.pallas import tpu as pltpu
from jax.experimental.pallas import tpu_sc as plsc

_VOCAB = 1000000
_DIM = 128
_BATCH = 4096
_HIST = 200
_IDS = _BATCH * _HIST

_BB = 64
_NCHUNK = 4
_CB = _BATCH // _NCHUNK
_CIDS = _CB * _HIST
_TILES = 32
_GW = 128
_GPT = _CIDS // (_TILES * _GW)


def _gather(table, ids3):
    mesh = plsc.VectorSubcoreMesh(core_axis_name="core",
                                  subcore_axis_name="subcore")

    _NBUF = 5
    _LAG = 2

    @functools.partial(
        pl.kernel,
        out_type=jax.ShapeDtypeStruct((_CIDS, _DIM), jnp.float32),
        mesh=mesh,
        scratch_types=[
            pltpu.VMEM((_GPT, _GW), jnp.int32),
            pltpu.VMEM((_NBUF, _GW, _DIM), jnp.float32),
            pltpu.SemaphoreType.DMA((_NBUF,)),
            pltpu.SemaphoreType.DMA((_NBUF,)),
        ],
    )
    def sc_kernel(table_hbm, ids_hbm, out_hbm, idx_v, buf, gsem, osem):
        wid = (jax.lax.axis_index("subcore") * 2
               + jax.lax.axis_index("core"))
        pltpu.sync_copy(ids_hbm.at[wid], idx_v)
        base = wid * (_GPT * _GW)

        for j in range(_GPT + _LAG):
            b = j % _NBUF
            if j < _GPT:
                if j >= _NBUF:
                    pltpu.make_async_copy(
                        buf.at[b], out_hbm.at[pl.ds(base, _GW)],
                        osem.at[b],
                    ).wait()
                pltpu.async_copy(
                    table_hbm.at[idx_v.at[j]], buf.at[b], gsem.at[b])
            if j >= _LAG:
                jj = j - _LAG
                bb = jj % _NBUF
                pltpu.make_async_copy(
                    table_hbm.at[idx_v.at[0]], buf.at[bb], gsem.at[bb],
                ).wait()
                pltpu.async_copy(
                    buf.at[bb],
                    out_hbm.at[pl.ds(base + jj * _GW, _GW)],
                    osem.at[bb],
                )
        for j in range(_GPT - _NBUF, _GPT):
            b = j % _NBUF
            pltpu.make_async_copy(
                buf.at[b], out_hbm.at[pl.ds(base, _GW)], osem.at[b],
            ).wait()

    return sc_kernel(table, ids3)


def _mlp_body(e_ref, w1t_ref, b1_ref, w2t_ref, b2_ref, s_ref, o_ref):
    h = jnp.dot(e_ref[...].astype(jnp.bfloat16), w1t_ref[...],
                preferred_element_type=jnp.float32) + b1_ref[...]
    h = jnp.maximum(h, 0.0)
    hm = jnp.dot(s_ref[...], h.astype(jnp.bfloat16),
                 preferred_element_type=jnp.float32)
    o_ref[...] = jnp.dot(hm, w2t_ref[...],
                         preferred_element_type=jnp.float32) + b2_ref[...]


def _mlp(embeds, w1t, b1, w2t, b2, seg):
    grid = _CB // _BB
    return pl.pallas_call(
        _mlp_body,
        grid=(grid,),
        in_specs=[
            pl.BlockSpec((_BB * _HIST, _DIM), lambda i: (i, 0)),
            pl.BlockSpec((_DIM, _DIM), lambda i: (0, 0)),
            pl.BlockSpec((1, _DIM), lambda i: (0, 0)),
            pl.BlockSpec((_DIM, _DIM), lambda i: (0, 0)),
            pl.BlockSpec((1, _DIM), lambda i: (0, 0)),
            pl.BlockSpec((_BB, _BB * _HIST), lambda i: (0, 0)),
        ],
        out_specs=pl.BlockSpec((_BB, _DIM), lambda i: (i, 0)),
        out_shape=jax.ShapeDtypeStruct((_CB, _DIM), jnp.float32),
    )(embeds, w1t, b1, w2t, b2, seg)


def kernel(input_ids, table, W1, b1, W2, b2):
    flat_ids = input_ids.reshape(_NCHUNK, _TILES, _GPT, _GW).astype(jnp.int32)
    col = jax.lax.broadcasted_iota(jnp.int32, (_BB, _BB * _HIST), 1)
    row = jax.lax.broadcasted_iota(jnp.int32, (_BB, _BB * _HIST), 0)
    seg = jnp.where(col // _HIST == row, jnp.float32(1.0),
                    jnp.float32(0.0)).astype(jnp.bfloat16)
    w1t, w2t = W1.T.astype(jnp.bfloat16), W2.T * jnp.float32(1.0 / _HIST)
    b1r, b2r = b1.reshape(1, _DIM), b2.reshape(1, _DIM)
    outs = []
    for c in range(_NCHUNK):
        emb_c = _gather(table, flat_ids[c])
        outs.append(_mlp(emb_c, w1t, b1r, w2t, b2r, seg))
    return jnp.concatenate(outs, axis=0)
